# baseline (device time: 34866 ns/iter reference)
import jax
import jax.numpy as jnp
from jax import lax
from jax.experimental import pallas as pl
from jax.experimental.pallas import tpu as pltpu

N_DEV = 4
DH = 64


def kernel(x, Wq, Wo, K_ext, V_ext):
    B, Sq, D = x.shape
    Hq_per = Wq.shape[1] // DH
    bf16 = jnp.bfloat16
    f32 = jnp.float32

    i = lax.axis_index("i")
    K_loc = lax.dynamic_slice_in_dim(K_ext, i * Hq_per, Hq_per, axis=2)
    V_loc = lax.dynamic_slice_in_dim(V_ext, i * Hq_per, Hq_per, axis=2)
    K_loc = K_loc.transpose(0, 2, 1, 3).astype(bf16)
    V_loc = V_loc.transpose(0, 2, 1, 3).astype(bf16)
    Wq_r = (Wq.reshape(D, Hq_per, DH) * 0.125).transpose(1, 0, 2).astype(bf16)
    Wo_r = Wo.reshape(Hq_per, DH, D).astype(bf16)

    def body(x_ref, wq_ref, wo_ref, k_ref, v_ref, out_ref,
             xbufA, xbufB, accA, accB,
             sendA, sendB, recvA, recvB,
             agA_send, agA_recv, agB_send, agB_recv,
             rsA_send, rsA_recv, rsB_send, rsB_recv):
        my = lax.axis_index("i")
        left = (my - 1) % N_DEV
        right = (my + 1) % N_DEV

        rings = (
            (xbufA, accA, sendA, recvA, agA_send, agA_recv,
             rsA_send, rsA_recv, right),
            (xbufB, accB, sendB, recvB, agB_send, agB_recv,
             rsB_send, rsB_recv, left),
        )

        barrier = pltpu.get_barrier_semaphore()
        for nbr in (left, right):
            pl.semaphore_signal(barrier, inc=1, device_id=(nbr,),
                                device_id_type=pl.DeviceIdType.MESH)
        pl.semaphore_wait(barrier, 2)

        xbufA[0] = x_ref[0].astype(bf16)
        xbufB[0] = x_ref[1].astype(bf16)

        def ag_rdma(r, h):
            xb, tgt = rings[r][0], rings[r][8]
            return pltpu.make_async_remote_copy(
                src_ref=xb.at[h], dst_ref=xb.at[h + 1],
                send_sem=rings[r][4].at[h], recv_sem=rings[r][5].at[h],
                device_id=(tgt,), device_id_type=pl.DeviceIdType.MESH)

        def rs_rdma(r, t):
            return pltpu.make_async_remote_copy(
                src_ref=rings[r][2].at[t], dst_ref=rings[r][3].at[t],
                send_sem=rings[r][6].at[t], recv_sem=rings[r][7].at[t],
                device_id=(rings[r][8],), device_id_type=pl.DeviceIdType.MESH)

        def compute_pair(s):
            xs = jnp.concatenate([xbufA[s], xbufB[s]], axis=0)
            part = [None, None]
            for h in range(Hq_per):
                qh = jnp.dot(xs, wq_ref[h],
                             preferred_element_type=f32).astype(bf16)
                for r in (0, 1):
                    q = qh[r * Sq:(r + 1) * Sq]
                    sc = lax.dot_general(
                        q, k_ref[r, h], (((1,), (1,)), ((), ())),
                        preferred_element_type=f32)
                    p = jnp.exp(sc)
                    d = jnp.sum(p, axis=1, keepdims=True)
                    o = jnp.dot(p.astype(bf16), v_ref[r, h],
                                preferred_element_type=f32)
                    ob = (o * (1.0 / d)).astype(bf16)
                    c = jnp.dot(ob, wo_ref[h], preferred_element_type=f32)
                    part[r] = c if h == 0 else part[r] + c
            accA[s] = part[0]
            accB[s] = part[1]

        def rs_payload(r, t):
            acc_, send_, recv_ = rings[r][1], rings[r][2], rings[r][3]
            val = acc_[t + 1]
            if t > 0:
                val = val + recv_[t - 1].astype(f32)
            send_[t] = val.astype(bf16)

        ag0 = [ag_rdma(r, 0) for r in (0, 1)]
        for d in ag0:
            d.start()
        compute_pair(0)
        for d in ag0:
            d.wait()

        ag1 = [ag_rdma(r, 1) for r in (0, 1)]
        for d in ag1:
            d.start()
        compute_pair(1)
        rs0 = [rs_rdma(r, 0) for r in (0, 1)]
        for r in (0, 1):
            rs_payload(r, 0)
            rs0[r].start()
        for d in ag1:
            d.wait()

        ag2 = [ag_rdma(r, 2) for r in (0, 1)]
        for d in ag2:
            d.start()
        compute_pair(2)
        for d in rs0:
            d.wait()
        rs1 = [rs_rdma(r, 1) for r in (0, 1)]
        for r in (0, 1):
            rs_payload(r, 1)
            rs1[r].start()
        for d in ag2:
            d.wait()

        compute_pair(3)
        for d in rs1:
            d.wait()
        rs2 = [rs_rdma(r, 2) for r in (0, 1)]
        for r in (0, 1):
            rs_payload(r, 2)
            rs2[r].start()
        for d in rs2:
            d.wait()

        out_ref[0] = accA[0] + recvA[N_DEV - 2].astype(f32)
        out_ref[1] = accB[0] + recvB[N_DEV - 2].astype(f32)

    return pl.pallas_call(
        body,
        out_shape=jax.ShapeDtypeStruct((B, Sq, D), f32),
        in_specs=[pl.BlockSpec(memory_space=pltpu.VMEM)] * 5,
        out_specs=pl.BlockSpec(memory_space=pltpu.VMEM),
        scratch_shapes=[
            pltpu.VMEM((N_DEV, Sq, D), bf16),
            pltpu.VMEM((N_DEV, Sq, D), bf16),
            pltpu.VMEM((N_DEV, Sq, D), f32),
            pltpu.VMEM((N_DEV, Sq, D), f32),
            pltpu.VMEM((N_DEV - 1, Sq, D), bf16),
            pltpu.VMEM((N_DEV - 1, Sq, D), bf16),
            pltpu.VMEM((N_DEV - 1, Sq, D), bf16),
            pltpu.VMEM((N_DEV - 1, Sq, D), bf16),
            pltpu.SemaphoreType.DMA((N_DEV - 1,)),
            pltpu.SemaphoreType.DMA((N_DEV - 1,)),
            pltpu.SemaphoreType.DMA((N_DEV - 1,)),
            pltpu.SemaphoreType.DMA((N_DEV - 1,)),
            pltpu.SemaphoreType.DMA((N_DEV - 1,)),
            pltpu.SemaphoreType.DMA((N_DEV - 1,)),
            pltpu.SemaphoreType.DMA((N_DEV - 1,)),
            pltpu.SemaphoreType.DMA((N_DEV - 1,)),
        ],
        compiler_params=pltpu.CompilerParams(collective_id=0),
    )(x, Wq_r, Wo_r, K_loc, V_loc)


# device time: 26676 ns/iter; 1.3070x vs baseline; 1.3070x over previous
import jax
import jax.numpy as jnp
import numpy as np
from jax import lax
from jax.experimental import pallas as pl
from jax.experimental.pallas import tpu as pltpu

N_DEV = 4
DH = 64


def kernel(x, Wq, Wo, K_ext, V_ext):
    B, Sq, D = x.shape
    Hq_per = Wq.shape[1] // DH
    Skv = K_ext.shape[1]
    bf16 = jnp.bfloat16
    f32 = jnp.float32

    i = lax.axis_index("i")
    K_loc = lax.dynamic_slice_in_dim(K_ext, i * Hq_per, Hq_per, axis=2)
    V_loc = lax.dynamic_slice_in_dim(V_ext, i * Hq_per, Hq_per, axis=2)
    K_loc = K_loc.transpose(0, 2, 1, 3).astype(bf16)
    V_loc = V_loc.transpose(0, 2, 1, 3).astype(bf16)
    eye = jnp.asarray(np.eye(Hq_per), bf16)
    KBD = jnp.einsum('bhsd,hg->bhdgs', K_loc, eye).reshape(
        B, Hq_per * DH, Hq_per * Skv)
    VST = jnp.einsum('bhsd,hg->bhsgd', V_loc, eye).reshape(
        B, Hq_per * Skv, Hq_per * DH)
    SUMB = jnp.asarray(np.kron(np.eye(Hq_per), np.ones((Skv, DH))), bf16)
    Wq_b = (Wq * 0.125).astype(bf16)
    Wo_b = Wo.astype(bf16)

    def body(x_ref, wq_ref, wo_ref, kbd_ref, vst_ref, sumb_ref, out_ref,
             xbufA, xbufB, accA, accB,
             sendA, sendB, recvA, recvB,
             agA_send, agA_recv, agB_send, agB_recv,
             rsA_send, rsA_recv, rsB_send, rsB_recv):
        my = lax.axis_index("i")
        left = (my - 1) % N_DEV
        right = (my + 1) % N_DEV

        rings = (
            (xbufA, accA, sendA, recvA, agA_send, agA_recv,
             rsA_send, rsA_recv, right),
            (xbufB, accB, sendB, recvB, agB_send, agB_recv,
             rsB_send, rsB_recv, left),
        )

        barrier = pltpu.get_barrier_semaphore()
        for nbr in (left, right):
            pl.semaphore_signal(barrier, inc=1, device_id=(nbr,),
                                device_id_type=pl.DeviceIdType.MESH)
        pl.semaphore_wait(barrier, 2)

        xbufA[0] = x_ref[0].astype(bf16)
        xbufB[0] = x_ref[1].astype(bf16)

        def ag_rdma(r, h):
            xb, tgt = rings[r][0], rings[r][8]
            return pltpu.make_async_remote_copy(
                src_ref=xb.at[h], dst_ref=xb.at[h + 1],
                send_sem=rings[r][4].at[h], recv_sem=rings[r][5].at[h],
                device_id=(tgt,), device_id_type=pl.DeviceIdType.MESH)

        def rs_rdma(r, t):
            return pltpu.make_async_remote_copy(
                src_ref=rings[r][2].at[t], dst_ref=rings[r][3].at[t],
                send_sem=rings[r][6].at[t], recv_sem=rings[r][7].at[t],
                device_id=(rings[r][8],), device_id_type=pl.DeviceIdType.MESH)

        def compute_pair(s):
            xs = jnp.concatenate([xbufA[s], xbufB[s]], axis=0)
            q_bf = jnp.dot(xs, wq_ref[...],
                           preferred_element_type=f32).astype(bf16)
            onorm = [None, None]
            for r in (0, 1):
                q = q_bf[r * Sq:(r + 1) * Sq]
                sc = jnp.dot(q, kbd_ref[r],
                             preferred_element_type=f32)
                p = jnp.exp(sc).astype(bf16)
                o = jnp.dot(p, vst_ref[r],
                            preferred_element_type=f32)
                den = jnp.dot(p, sumb_ref[...],
                              preferred_element_type=f32)
                onorm[r] = (o / den).astype(bf16)
            ostack = jnp.concatenate(onorm, axis=0)
            part = jnp.dot(ostack, wo_ref[...], preferred_element_type=f32)
            accA[s] = part[:Sq]
            accB[s] = part[Sq:]

        def rs_payload(r, t):
            acc_, send_, recv_ = rings[r][1], rings[r][2], rings[r][3]
            val = acc_[t + 1]
            if t > 0:
                val = val + recv_[t - 1].astype(f32)
            send_[t] = val.astype(bf16)

        ag0 = [ag_rdma(r, 0) for r in (0, 1)]
        for d in ag0:
            d.start()
        compute_pair(0)
        for d in ag0:
            d.wait()

        ag1 = [ag_rdma(r, 1) for r in (0, 1)]
        for d in ag1:
            d.start()
        compute_pair(1)
        rs0 = [rs_rdma(r, 0) for r in (0, 1)]
        for r in (0, 1):
            rs_payload(r, 0)
            rs0[r].start()
        for d in ag1:
            d.wait()

        ag2 = [ag_rdma(r, 2) for r in (0, 1)]
        for d in ag2:
            d.start()
        compute_pair(2)
        for d in rs0:
            d.wait()
        rs1 = [rs_rdma(r, 1) for r in (0, 1)]
        for r in (0, 1):
            rs_payload(r, 1)
            rs1[r].start()
        for d in ag2:
            d.wait()

        compute_pair(3)
        for d in rs1:
            d.wait()
        rs2 = [rs_rdma(r, 2) for r in (0, 1)]
        for r in (0, 1):
            rs_payload(r, 2)
            rs2[r].start()
        for d in rs2:
            d.wait()

        out_ref[0] = accA[0] + recvA[N_DEV - 2].astype(f32)
        out_ref[1] = accB[0] + recvB[N_DEV - 2].astype(f32)

    return pl.pallas_call(
        body,
        out_shape=jax.ShapeDtypeStruct((B, Sq, D), f32),
        in_specs=[pl.BlockSpec(memory_space=pltpu.VMEM)] * 6,
        out_specs=pl.BlockSpec(memory_space=pltpu.VMEM),
        scratch_shapes=[
            pltpu.VMEM((N_DEV, Sq, D), bf16),
            pltpu.VMEM((N_DEV, Sq, D), bf16),
            pltpu.VMEM((N_DEV, Sq, D), f32),
            pltpu.VMEM((N_DEV, Sq, D), f32),
            pltpu.VMEM((N_DEV - 1, Sq, D), bf16),
            pltpu.VMEM((N_DEV - 1, Sq, D), bf16),
            pltpu.VMEM((N_DEV - 1, Sq, D), bf16),
            pltpu.VMEM((N_DEV - 1, Sq, D), bf16),
            pltpu.SemaphoreType.DMA((N_DEV - 1,)),
            pltpu.SemaphoreType.DMA((N_DEV - 1,)),
            pltpu.SemaphoreType.DMA((N_DEV - 1,)),
            pltpu.SemaphoreType.DMA((N_DEV - 1,)),
            pltpu.SemaphoreType.DMA((N_DEV - 1,)),
            pltpu.SemaphoreType.DMA((N_DEV - 1,)),
            pltpu.SemaphoreType.DMA((N_DEV - 1,)),
            pltpu.SemaphoreType.DMA((N_DEV - 1,)),
        ],
        compiler_params=pltpu.CompilerParams(collective_id=0),
    )(x, Wq_b, Wo_b, KBD, VST, SUMB)
